# baseline (speedup 1.0000x reference)
import functools

import jax
import jax.numpy as jnp
from jax import lax
from jax.experimental import pallas as pl
from jax.experimental.pallas import tpu as pltpu
from jax.experimental.pallas import tpu_sc as plsc

B = 16384
NEG = 20
D = 64
L = 16
NC = 2
NS = 16
NW = NC * NS
BPW = B // NW
CHUNK = 32
NCHUNK = BPW // CHUNK
NGIDX = CHUNK * NEG // 128


def _sc_dots(center_h, outside_h, neg_h, wc2_h, wo2_h, uo_out, uk_out,
             idx_c, idx_o, idx_n, scb, sob, snb, c_rows, o_rows, n_rows,
             uo_buf, uk_buf, sem):
    wid = lax.axis_index("s") * NC + lax.axis_index("c")
    base = wid * BPW
    pltpu.sync_copy(center_h.at[pl.ds(base, BPW)], idx_c.at[pl.ds(0, BPW)])
    pltpu.sync_copy(outside_h.at[pl.ds(base, BPW)], idx_o.at[pl.ds(0, BPW)])
    pltpu.sync_copy(neg_h.at[pl.ds(base * NEG, BPW * NEG)], idx_n)

    def chunk_body(t, _):
        for v in range(CHUNK // L):
            scb[pl.ds(v * L, L)] = idx_c[pl.ds(t * CHUNK + v * L, L)] >> 1
            sob[pl.ds(v * L, L)] = idx_o[pl.ds(t * CHUNK + v * L, L)] >> 1
        for v in range(CHUNK * NEG // L):
            snb[pl.ds(v * L, L)] = (
                idx_n[pl.ds(t * CHUNK * NEG + v * L, L)] >> 1)
        cps = [pltpu.async_copy(wc2_h.at[scb], c_rows, sem),
               pltpu.async_copy(wo2_h.at[sob], o_rows, sem)]
        for j in range(NGIDX):
            cps.append(pltpu.async_copy(wo2_h.at[snb.at[pl.ds(j * 128, 128)]],
                                        n_rows.at[pl.ds(j * 128, 128)], sem))
        for cp in cps:
            cp.wait()

        def item_body(m, _):
            gi = t * CHUNK + m
            offc = (idx_c[pl.ds(gi, L)][0] & 1) * D
            offo = (idx_o[pl.ds(gi, L)][0] & 1) * D
            nv0 = (idx_n[pl.ds(gi * NEG, L)] & 1) * D
            nv1 = (idx_n[pl.ds(gi * NEG + NEG - L, L)] & 1) * D
            cj = [c_rows[m, pl.ds(offc + j * L, L)] for j in range(D // L)]
            oj = [o_rows[m, pl.ds(offo + j * L, L)] for j in range(D // L)]
            p = cj[0] * oj[0]
            for j in range(1, D // L):
                p = p + cj[j] * oj[j]
            sj = [n_rows[m * NEG, pl.ds(nv0[0] + j * L, L)]
                  for j in range(D // L)]
            for k in range(1, NEG):
                offn = nv0[k] if k < L else nv1[k - (NEG - L)]
                for j in range(D // L):
                    sj[j] = sj[j] + n_rows[m * NEG + k,
                                           pl.ds(offn + j * L, L)]
            q = cj[0] * sj[0]
            for j in range(1, D // L):
                q = q + cj[j] * sj[j]
            uo_buf[pl.ds(gi * L, L)] = p
            uk_buf[pl.ds(gi * L, L)] = -q
            return 0

        lax.fori_loop(0, CHUNK, item_body, 0)
        return 0

    lax.fori_loop(0, NCHUNK, chunk_body, 0)
    pltpu.sync_copy(uo_buf, uo_out.at[pl.ds(base * L, BPW * L)])
    pltpu.sync_copy(uk_buf, uk_out.at[pl.ds(base * L, BPW * L)])


@functools.partial(
    pl.kernel,
    mesh=plsc.VectorSubcoreMesh(core_axis_name="c", subcore_axis_name="s"),
    out_type=[jax.ShapeDtypeStruct((B * L,), jnp.float32),
              jax.ShapeDtypeStruct((B * L,), jnp.float32)],
    scratch_types=[
        pltpu.VMEM((BPW + L,), jnp.int32),
        pltpu.VMEM((BPW + L,), jnp.int32),
        pltpu.VMEM((BPW * NEG,), jnp.int32),
        pltpu.VMEM((CHUNK,), jnp.int32),
        pltpu.VMEM((CHUNK,), jnp.int32),
        pltpu.VMEM((CHUNK * NEG,), jnp.int32),
        pltpu.VMEM((CHUNK, 2 * D), jnp.float32),
        pltpu.VMEM((CHUNK, 2 * D), jnp.float32),
        pltpu.VMEM((CHUNK * NEG, 2 * D), jnp.float32),
        pltpu.VMEM((BPW * L,), jnp.float32),
        pltpu.VMEM((BPW * L,), jnp.float32),
        pltpu.SemaphoreType.DMA,
    ],
)
def _sc_kernel(center_h, outside_h, neg_h, wc2_h, wo2_h, uo_out, uk_out,
               idx_c, idx_o, idx_n, scb, sob, snb, c_rows, o_rows, n_rows,
               uo_buf, uk_buf, sem):
    _sc_dots(center_h, outside_h, neg_h, wc2_h, wo2_h, uo_out, uk_out,
             idx_c, idx_o, idx_n, scb, sob, snb, c_rows, o_rows, n_rows,
             uo_buf, uk_buf, sem)


def _loss_body(uo_ref, uk_ref, out_ref):
    a = uo_ref[...].reshape(B // 8, 8, L).sum(axis=-1)
    b = uk_ref[...].reshape(B // 8, 8, L).sum(axis=-1)

    def logsig(x):
        return jnp.minimum(x, 0.0) - jnp.log(1.0 + jnp.exp(-jnp.abs(x)))

    out_ref[...] = jnp.full((1, 1), -jnp.sum(logsig(a) + logsig(b)) / B)


def kernel(center, outside, negative, W_center, W_outside):
    center = center.reshape(B)
    outside = outside.reshape(B)
    neg = negative.reshape(B * NEG)
    wc2 = W_center.reshape(-1, 2 * D)
    wo2 = W_outside.reshape(-1, 2 * D)
    uo, uk = _sc_kernel(center, outside, neg, wc2, wo2)
    loss = pl.pallas_call(
        _loss_body,
        out_shape=jax.ShapeDtypeStruct((1, 1), jnp.float32),
    )(uo.reshape(B // 8, 8 * L), uk.reshape(B // 8, 8 * L))
    return loss[0, 0]
